# baseline (speedup 1.0000x reference)
import functools

import jax
import jax.numpy as jnp
from jax import lax
from jax.experimental import pallas as pl
from jax.experimental.pallas import tpu as pltpu
from jax.experimental.pallas import tpu_sc as plsc

_NUM_EMB = 1000000
_D = 32
_B = 16384
_H = 200

_TOT = _B * _H
_NC, _NS = 2, 16
_NW = _NC * _NS
_BT = _B // 128
_NTILE = _H * _BT
_PER_W = _NTILE // _NW
_NBUF = 4
_ROUNDS = _PER_W // _NBUF
_CHUNK = 80
_CIDX = _CHUNK * 128


@functools.partial(
    pl.kernel,
    mesh=plsc.VectorSubcoreMesh(core_axis_name="c", subcore_axis_name="s"),
    out_type=jax.ShapeDtypeStruct((_H, _D // 8, _BT, 8, 128), jnp.float32),
    compiler_params=pltpu.CompilerParams(
        use_tc_tiling_on_sc=False, needs_layout_passes=False
    ),
    scratch_types=[
        pltpu.VMEM((2 * _CIDX,), jnp.int32),
        pltpu.VMEM((_NBUF, 128, _D), jnp.float32),
        pltpu.VMEM((_NBUF, _D, 128), jnp.float32),
    ]
    + [pltpu.SemaphoreType.DMA] * (2 * _NBUF),
)
def _emb_gather(idx_hbm, tab_hbm, out_hbm, idx_v, rows_v, trans_v, *sems):
    sem_g = sems[:_NBUF]
    sem_o = sems[_NBUF:]
    wid = lax.axis_index("s") * _NC + lax.axis_index("c")
    base = wid * _PER_W

    row_vecs = [jnp.arange(16, dtype=jnp.int32) + 16 * k for k in range(8)]
    col_vecs = [jnp.full((16,), f, dtype=jnp.int32) for f in range(_D)]

    def load_chunk(g):
        c = g // _CHUNK
        src = pl.multiple_of((base + c * _CHUNK) * 128, 1024)
        dst = pl.multiple_of((c % 2) * _CIDX, 1024)
        pltpu.sync_copy(
            idx_hbm.at[pl.ds(src, _CIDX)], idx_v.at[pl.ds(dst, _CIDX)]
        )

    def fire_gather(g, b):
        off = pl.multiple_of((g % (2 * _CHUNK)) * 128, 128)
        pltpu.async_copy(
            tab_hbm.at[idx_v.at[pl.ds(off, 128)]], rows_v.at[b], sem_g[b]
        )

    def drain_gather(b):
        pltpu.make_async_copy(
            tab_hbm.at[idx_v.at[pl.ds(0, 128)]], rows_v.at[b], sem_g[b]
        ).wait()

    def transpose_block(b):
        for f in range(_D):
            for k in range(8):
                vals = plsc.load_gather(
                    rows_v.at[b], [row_vecs[k], col_vecs[f]]
                )
                trans_v[b, f, pl.ds(16 * k, 16)] = vals

    def fire_stores(gp, b):
        g_abs = base + gp
        h = g_abs // _BT
        tc = g_abs % _BT
        for tr in range(_D // 8):
            pltpu.async_copy(
                trans_v.at[b, pl.ds(8 * tr, 8)],
                out_hbm.at[h, tr, tc],
                sem_o[b],
            )

    def drain_stores(b):
        for _ in range(_D // 8):
            pltpu.make_async_copy(
                trans_v.at[b, pl.ds(0, 8)], out_hbm.at[0, 0, 0], sem_o[b]
            ).wait()

    def body(r, carry):
        g0 = r * _NBUF

        @pl.when(g0 % _CHUNK == 0)
        def _():
            load_chunk(g0)

        for vb in range(_NBUF):
            g = g0 + vb
            gp = g - _NBUF

            @pl.when(g >= 2 * _NBUF)
            def _():
                drain_stores(vb)

            @pl.when(g >= _NBUF)
            def _():
                drain_gather(vb)
                transpose_block(vb)
                fire_stores(gp, vb)

            fire_gather(g, vb)
        return carry

    lax.fori_loop(0, _ROUNDS, body, 0)

    for vb in range(_NBUF):
        gp = _PER_W - _NBUF + vb
        drain_stores(vb)
        drain_gather(vb)
        transpose_block(vb)
        fire_stores(gp, vb)
    for vb in range(_NBUF):
        drain_stores(vb)


def kernel(token_ids, weight):
    idx_hm = jnp.transpose(token_ids).reshape(_TOT)
    out5 = _emb_gather(idx_hm, weight)
    return out5.transpose(2, 4, 0, 1, 3).reshape(_B, _H, _D)


# speedup vs baseline: 1.3296x; 1.3296x over previous
import functools

import jax
import jax.numpy as jnp
from jax import lax
from jax.experimental import pallas as pl
from jax.experimental.pallas import tpu as pltpu
from jax.experimental.pallas import tpu_sc as plsc

_NUM_EMB = 1000000
_D = 32
_B = 16384
_H = 200

_TOT = _B * _H
_NC, _NS = 2, 16
_NW = _NC * _NS
_BT = _B // 128
_NTILE = _H * _BT
_PER_W = _NTILE // _NW
_NBUF = 4
_ROUNDS = _PER_W // _NBUF
_CHUNK = 80
_CIDX = _CHUNK * 128


@functools.partial(
    pl.kernel,
    mesh=plsc.VectorSubcoreMesh(core_axis_name="c", subcore_axis_name="s"),
    out_type=jax.ShapeDtypeStruct((_H, _D // 8, _BT, 8 * 128), jnp.float32),
    compiler_params=pltpu.CompilerParams(
        use_tc_tiling_on_sc=False, needs_layout_passes=False
    ),
    scratch_types=[
        pltpu.VMEM((2 * _CIDX,), jnp.int32),
        pltpu.VMEM((_NBUF, 128, _D), jnp.float32),
        pltpu.VMEM((_NBUF, _D * 128), jnp.float32),
    ]
    + [pltpu.SemaphoreType.DMA] * (2 * _NBUF),
)
def _emb_gather(idx_hbm, tab_hbm, out_hbm, idx_v, rows_v, trans_v, *sems):
    sem_g = sems[:_NBUF]
    sem_o = sems[_NBUF:]
    wid = lax.axis_index("s") * _NC + lax.axis_index("c")
    base = wid * _PER_W

    v_lo = jnp.arange(16, dtype=jnp.int32) * 128
    v_hi = v_lo + 16 * 128

    def load_chunk(g):
        c = g // _CHUNK
        src = pl.multiple_of((base + c * _CHUNK) * 128, 1024)
        dst = pl.multiple_of((c % 2) * _CIDX, 1024)
        pltpu.sync_copy(
            idx_hbm.at[pl.ds(src, _CIDX)], idx_v.at[pl.ds(dst, _CIDX)]
        )

    def fire_gather(g, b):
        off = pl.multiple_of((g % (2 * _CHUNK)) * 128, 128)
        pltpu.async_copy(
            tab_hbm.at[idx_v.at[pl.ds(off, 128)]], rows_v.at[b], sem_g[b]
        )

    def drain_gather(b):
        pltpu.make_async_copy(
            tab_hbm.at[idx_v.at[pl.ds(0, 128)]], rows_v.at[b], sem_g[b]
        ).wait()

    def transpose_block(b):
        tr_ref = trans_v.at[b]
        for j in range(128):
            lo = rows_v[b, j, pl.ds(0, 16)]
            hi = rows_v[b, j, pl.ds(16, 16)]
            plsc.store_scatter(tr_ref, [v_lo + j], lo)
            plsc.store_scatter(tr_ref, [v_hi + j], hi)

    def fire_stores(gp, b):
        g_abs = base + gp
        h = g_abs // _BT
        tc = g_abs % _BT
        for tr in range(_D // 8):
            pltpu.async_copy(
                trans_v.at[b, pl.ds(tr * 1024, 1024)],
                out_hbm.at[h, tr, tc],
                sem_o[b],
            )

    def drain_stores(b):
        for _ in range(_D // 8):
            pltpu.make_async_copy(
                trans_v.at[b, pl.ds(0, 1024)], out_hbm.at[0, 0, 0], sem_o[b]
            ).wait()

    def body(r, carry):
        g0 = r * _NBUF

        @pl.when(g0 % _CHUNK == 0)
        def _():
            load_chunk(g0)

        for vb in range(_NBUF):
            g = g0 + vb
            gp = g - _NBUF

            @pl.when(g >= 2 * _NBUF)
            def _():
                drain_stores(vb)

            @pl.when(g >= _NBUF)
            def _():
                drain_gather(vb)
                transpose_block(vb)
                fire_stores(gp, vb)

            fire_gather(g, vb)
        return carry

    lax.fori_loop(0, _ROUNDS, body, 0)

    for vb in range(_NBUF):
        gp = _PER_W - _NBUF + vb
        drain_stores(vb)
        drain_gather(vb)
        transpose_block(vb)
        fire_stores(gp, vb)
    for vb in range(_NBUF):
        drain_stores(vb)


def kernel(token_ids, weight):
    idx_hm = jnp.transpose(token_ids).reshape(_TOT)
    out5 = _emb_gather(idx_hm, weight)
    out6 = out5.reshape(_H, _D // 8, _BT, 8, 128)
    return out6.transpose(2, 4, 0, 1, 3).reshape(_B, _H, _D)


# speedup vs baseline: 1.9066x; 1.4340x over previous
import functools

import jax
import jax.numpy as jnp
from jax import lax
from jax.experimental import pallas as pl
from jax.experimental.pallas import tpu as pltpu
from jax.experimental.pallas import tpu_sc as plsc

_NUM_EMB = 1000000
_D = 32
_B = 16384
_H = 200

_TOT = _B * _H
_NC, _NS = 2, 16
_NW = _NC * _NS
_BT = _B // 128
_NTILE = _H * _BT
_PER_W = _NTILE // _NW
_NBUF = 4
_ROUNDS = _PER_W // _NBUF
_CHUNK = 80
_CIDX = _CHUNK * 128


@functools.partial(
    pl.kernel,
    mesh=plsc.VectorSubcoreMesh(core_axis_name="c", subcore_axis_name="s"),
    out_type=jax.ShapeDtypeStruct((_H, _D // 8, _BT, 8, 128), jnp.float32),
    compiler_params=pltpu.CompilerParams(
        use_tc_tiling_on_sc=False, needs_layout_passes=False
    ),
    scratch_types=[
        pltpu.VMEM((2 * _CIDX,), jnp.int32),
        pltpu.VMEM((_NBUF, 128, _D), jnp.float32),
        pltpu.VMEM((_NBUF, _D, 137), jnp.float32),
    ]
    + [pltpu.SemaphoreType.DMA] * (2 * _NBUF),
)
def _emb_gather(idx_hbm, tab_hbm, out_hbm, idx_v, rows_v, trans_v, *sems):
    sem_g = sems[:_NBUF]
    sem_o = sems[_NBUF:]
    wid = lax.axis_index("s") * _NC + lax.axis_index("c")
    base = wid * _PER_W

    r_lo = jnp.arange(16, dtype=jnp.int32)
    r_hi = r_lo + 16

    def load_chunk(g):
        c = g // _CHUNK
        src = pl.multiple_of((base + c * _CHUNK) * 128, 1024)
        dst = pl.multiple_of((c % 2) * _CIDX, 1024)
        pltpu.sync_copy(
            idx_hbm.at[pl.ds(src, _CIDX)], idx_v.at[pl.ds(dst, _CIDX)]
        )

    def fire_gather(g, b):
        off = pl.multiple_of((g % (2 * _CHUNK)) * 128, 128)
        pltpu.async_copy(
            tab_hbm.at[idx_v.at[pl.ds(off, 128)]], rows_v.at[b], sem_g[b]
        )

    def drain_gather(b):
        pltpu.make_async_copy(
            tab_hbm.at[idx_v.at[pl.ds(0, 128)]], rows_v.at[b], sem_g[b]
        ).wait()

    def transpose_block(b):
        tr_ref = trans_v.at[b]
        for j in range(128):
            lo = rows_v[b, j, pl.ds(0, 16)]
            hi = rows_v[b, j, pl.ds(16, 16)]
            cj = jnp.full((16,), j, jnp.int32)
            plsc.store_scatter(tr_ref, [r_lo, cj], lo)
            plsc.store_scatter(tr_ref, [r_hi, cj], hi)

    def fire_stores(gp, b):
        g_abs = base + gp
        h = g_abs // _BT
        tc = g_abs % _BT
        for tr in range(_D // 8):
            pltpu.async_copy(
                trans_v.at[b, pl.ds(tr * 8, 8), pl.ds(0, 128)],
                out_hbm.at[h, tr, tc],
                sem_o[b],
            )

    def drain_stores(b):
        for _ in range(_D // 8):
            pltpu.make_async_copy(
                trans_v.at[b, pl.ds(0, 8), pl.ds(0, 128)],
                out_hbm.at[0, 0, 0],
                sem_o[b],
            ).wait()

    def body(r, carry):
        g0 = r * _NBUF

        @pl.when(g0 % _CHUNK == 0)
        def _():
            load_chunk(g0)

        for vb in range(_NBUF):
            g = g0 + vb
            gp = g - _NBUF

            @pl.when(g >= 2 * _NBUF)
            def _():
                drain_stores(vb)

            @pl.when(g >= _NBUF)
            def _():
                drain_gather(vb)
                transpose_block(vb)
                fire_stores(gp, vb)

            fire_gather(g, vb)
        return carry

    lax.fori_loop(0, _ROUNDS, body, 0)

    for vb in range(_NBUF):
        gp = _PER_W - _NBUF + vb
        drain_stores(vb)
        drain_gather(vb)
        transpose_block(vb)
        fire_stores(gp, vb)
    for vb in range(_NBUF):
        drain_stores(vb)


def kernel(token_ids, weight):
    idx_hm = jnp.transpose(token_ids).reshape(_TOT)
    out5 = _emb_gather(idx_hm, weight)
    return out5.transpose(2, 4, 0, 1, 3).reshape(_B, _H, _D)


# speedup vs baseline: 1.9091x; 1.0013x over previous
import functools

import jax
import jax.numpy as jnp
from jax import lax
from jax.experimental import pallas as pl
from jax.experimental.pallas import tpu as pltpu
from jax.experimental.pallas import tpu_sc as plsc

_NUM_EMB = 1000000
_D = 32
_B = 16384
_H = 200

_TOT = _B * _H
_NC, _NS = 2, 16
_NW = _NC * _NS
_BT = _B // 128
_NTILE = _H * _BT
_PER_W = _NTILE // _NW
_NBUF = 4
_ROUNDS = _PER_W // _NBUF
_CHUNK = 80
_CIDX = _CHUNK * 128


@functools.partial(
    pl.kernel,
    mesh=plsc.VectorSubcoreMesh(core_axis_name="c", subcore_axis_name="s"),
    out_type=jax.ShapeDtypeStruct((_H, _D // 8, _BT, 8, 128), jnp.float32),
    compiler_params=pltpu.CompilerParams(
        use_tc_tiling_on_sc=False, needs_layout_passes=False
    ),
    scratch_types=[
        pltpu.VMEM((2 * _CIDX,), jnp.int32),
        pltpu.VMEM((_NBUF, 128, _D), jnp.float32),
        pltpu.VMEM((_NBUF, _D, 137), jnp.float32),
    ]
    + [pltpu.SemaphoreType.DMA] * (2 * _NBUF),
)
def _emb_gather(idx_hbm, tab_hbm, out_hbm, idx_v, rows_v, trans_v, *sems):
    sem_g = sems[:_NBUF]
    sem_o = sems[_NBUF:]
    wid = lax.axis_index("s") * _NC + lax.axis_index("c")
    base = wid * _PER_W

    r_lo = jnp.arange(16, dtype=jnp.int32)
    r_hi = r_lo + 16

    def load_chunk(g):
        c = g // _CHUNK
        src = pl.multiple_of((base + c * _CHUNK) * 128, 1024)
        dst = pl.multiple_of((c % 2) * _CIDX, 1024)
        pltpu.sync_copy(
            idx_hbm.at[pl.ds(src, _CIDX)], idx_v.at[pl.ds(dst, _CIDX)]
        )

    def fire_gather(g, b):
        off = pl.multiple_of((g % (2 * _CHUNK)) * 128, 128)
        pltpu.async_copy(
            tab_hbm.at[idx_v.at[pl.ds(off, 128)]], rows_v.at[b], sem_g[b]
        )

    def drain_gather(b):
        pltpu.make_async_copy(
            tab_hbm.at[idx_v.at[pl.ds(0, 128)]], rows_v.at[b], sem_g[b]
        ).wait()

    def transpose_block(b, zero16):
        tr_ref = trans_v.at[b]
        cj = zero16
        one = zero16 + 1
        for j in range(128):
            lo = rows_v[b, j, pl.ds(0, 16)]
            hi = rows_v[b, j, pl.ds(16, 16)]
            plsc.store_scatter(tr_ref, [r_lo, cj], lo)
            plsc.store_scatter(tr_ref, [r_hi, cj], hi)
            cj = cj + one

    def fire_stores(gp, b):
        g_abs = base + gp
        h = g_abs // _BT
        tc = g_abs % _BT
        for tr in range(_D // 8):
            pltpu.async_copy(
                trans_v.at[b, pl.ds(tr * 8, 8), pl.ds(0, 128)],
                out_hbm.at[h, tr, tc],
                sem_o[b],
            )

    def drain_stores(b):
        for _ in range(_D // 8):
            pltpu.make_async_copy(
                trans_v.at[b, pl.ds(0, 8), pl.ds(0, 128)],
                out_hbm.at[0, 0, 0],
                sem_o[b],
            ).wait()

    def body(r, carry):
        g0 = r * _NBUF
        zero16 = jax.lax.broadcast(r * 0, (16,))

        @pl.when(g0 % _CHUNK == 0)
        def _():
            load_chunk(g0)

        for vb in range(_NBUF):
            g = g0 + vb
            gp = g - _NBUF

            @pl.when(g >= 2 * _NBUF)
            def _():
                drain_stores(vb)

            @pl.when(g >= _NBUF)
            def _():
                drain_gather(vb)
                transpose_block(vb, zero16)
                fire_stores(gp, vb)

            fire_gather(g, vb)
        return carry

    lax.fori_loop(0, _ROUNDS, body, 0)

    for vb in range(_NBUF):
        gp = _PER_W - _NBUF + vb
        drain_stores(vb)
        drain_gather(vb)
        transpose_block(vb, jnp.zeros((16,), jnp.int32))
        fire_stores(gp, vb)
    for vb in range(_NBUF):
        drain_stores(vb)


def kernel(token_ids, weight):
    idx_hm = jnp.transpose(token_ids).reshape(_TOT)
    out5 = _emb_gather(idx_hm, weight)
    return out5.transpose(2, 4, 0, 1, 3).reshape(_B, _H, _D)
